# Initial kernel scaffold; baseline (speedup 1.0000x reference)
import jax
import jax.numpy as jnp
from jax.experimental import pallas as pl


def kernel(x, edge_index, edge_attr, W0a, b0a, W0b, b0b, Wm1, bm1, Wm2, bm2, We1, be1, We2, be2, gamma, beta):
    raise NotImplementedError("write your pallas kernel here")


# speedup vs baseline: 3.2552x; 3.2552x over previous
import functools

import jax
import jax.numpy as jnp
from jax import lax
from jax.experimental import pallas as pl
from jax.experimental.pallas import tpu as pltpu
from jax.experimental.pallas import tpu_sc as plsc

N = 10000
E = 320000
H = 128
L = 3

NC, NS = 2, 16
EPC = E // NC
EPT = E // (NC * NS)
WIN = 80
NWIN = EPT // WIN


def _leaky(v):
    return jnp.where(v > 0, v, 0.1 * v)


def _node_mlp_body(x_ref, wa_ref, ba_ref, wb_ref, bb_ref, out_ref):
    t = jnp.dot(x_ref[...], wa_ref[...], preferred_element_type=jnp.float32)
    t = _leaky(t + ba_ref[...])
    out_ref[...] = (
        jnp.dot(t, wb_ref[...], preferred_element_type=jnp.float32) + bb_ref[...]
    )


_NB = 10
_BR = N // _NB


def _node_mlp(x, W0a, b0a, W0b, b0b):
    d_in = x.shape[1]
    return pl.pallas_call(
        _node_mlp_body,
        grid=(_NB,),
        in_specs=[
            pl.BlockSpec((_BR, d_in), lambda i: (i, 0)),
            pl.BlockSpec((d_in, H), lambda i: (0, 0)),
            pl.BlockSpec((1, H), lambda i: (0, 0)),
            pl.BlockSpec((H, H), lambda i: (0, 0)),
            pl.BlockSpec((1, H), lambda i: (0, 0)),
        ],
        out_specs=pl.BlockSpec((_BR, H), lambda i: (i, 0)),
        out_shape=jax.ShapeDtypeStruct((N, H), jnp.float32),
    )(x, W0a, b0a.reshape(1, H), W0b, b0b.reshape(1, H))


_EC = 8000


def _edge_mlp_body(ea_ref, w1_ref, b1_ref, w2_ref, b2_ref, out_ref):
    t = _leaky(
        jnp.dot(ea_ref[...], w1_ref[0], preferred_element_type=jnp.float32)
        + b1_ref[0]
    )
    out_ref[...] = (
        jnp.dot(t, w2_ref[0], preferred_element_type=jnp.float32) + b2_ref[0]
    )


def _edge_mlp(ea, We1, be1, We2, be2):
    d_e = ea.shape[1]
    return pl.pallas_call(
        _edge_mlp_body,
        grid=(L, E // _EC),
        in_specs=[
            pl.BlockSpec((_EC, d_e), lambda l, i: (i, 0)),
            pl.BlockSpec((1, d_e, H), lambda l, i: (l, 0, 0)),
            pl.BlockSpec((1, 1, H), lambda l, i: (l, 0, 0)),
            pl.BlockSpec((1, H, H), lambda l, i: (l, 0, 0)),
            pl.BlockSpec((1, 1, H), lambda l, i: (l, 0, 0)),
        ],
        out_specs=pl.BlockSpec((_EC, H), lambda l, i: (l * (E // _EC) + i, 0)),
        out_shape=jax.ShapeDtypeStruct((L * E, H), jnp.float32),
    )(ea, We1, be1, We2, be2)


ZR = 1000
NZT = N // ZR


def _sc_edge_agg_body(e_hbm, dst_hbm, zeros_hbm,
                      se_hbm,
                      didx, rows, acc_sh, sem):
    c = lax.axis_index("c")
    s = lax.axis_index("s")
    base = c * EPC + s * EPT
    r0 = s * ZR

    for l in range(L):
        @pl.when(s < NZT)
        def _():
            pltpu.sync_copy(zeros_hbm.at[pl.ds(r0, ZR)],
                            acc_sh.at[pl.ds(r0, ZR)])

        plsc.subcore_barrier()

        def win_body(w, carry):
            off = base + w * WIN
            pltpu.sync_copy(dst_hbm.at[pl.ds(off, WIN)], didx)
            pltpu.sync_copy(e_hbm.at[pl.ds(l * E + off, WIN)], rows)
            pltpu.sync_copy(rows, acc_sh.at[didx], add=True)
            return carry

        lax.fori_loop(0, NWIN, win_body, 0)
        plsc.subcore_barrier()

        @pl.when(s < NZT)
        def _():
            pltpu.sync_copy(acc_sh.at[pl.ds(r0, ZR)],
                            se_hbm.at[pl.ds((l * NC + c) * N + r0, ZR)])

        plsc.subcore_barrier()


def _sc_edge_agg(e, dst):
    zeros = jnp.zeros((N, H), jnp.float32)
    f = pl.kernel(
        _sc_edge_agg_body,
        out_type=jax.ShapeDtypeStruct((L * NC * N, H), jnp.float32),
        mesh=plsc.VectorSubcoreMesh(core_axis_name="c", subcore_axis_name="s"),
        scratch_types=[
            pltpu.VMEM((WIN,), jnp.int32),
            pltpu.VMEM((WIN, H), jnp.float32),
            pltpu.VMEM_SHARED((N, H), jnp.float32),
            pltpu.SemaphoreType.DMA,
        ],
    )
    return f(e, dst, zeros).reshape(L, NC, N, H)


def _sc_msg_agg_body(h_hbm, src_hbm, dst_hbm, zeros_hbm,
                     sh_hbm,
                     sidx, didx, rows, acc_sh, sem):
    c = lax.axis_index("c")
    s = lax.axis_index("s")
    base = c * EPC + s * EPT
    r0 = s * ZR

    @pl.when(s < NZT)
    def _():
        pltpu.sync_copy(zeros_hbm.at[pl.ds(r0, ZR)],
                        acc_sh.at[pl.ds(r0, ZR)])

    plsc.subcore_barrier()

    def win_body(w, carry):
        off = base + w * WIN
        pltpu.sync_copy(src_hbm.at[pl.ds(off, WIN)], sidx)
        pltpu.sync_copy(dst_hbm.at[pl.ds(off, WIN)], didx)
        pltpu.async_copy(h_hbm.at[sidx], rows, sem).wait()
        pltpu.sync_copy(rows, acc_sh.at[didx], add=True)
        return carry

    lax.fori_loop(0, NWIN, win_body, 0)
    plsc.subcore_barrier()

    @pl.when(s < NZT)
    def _():
        pltpu.sync_copy(acc_sh.at[pl.ds(r0, ZR)],
                        sh_hbm.at[pl.ds(c * N + r0, ZR)])


def _sc_msg_agg(h, src, dst):
    zeros = jnp.zeros((N, H), jnp.float32)
    f = pl.kernel(
        _sc_msg_agg_body,
        out_type=jax.ShapeDtypeStruct((NC * N, H), jnp.float32),
        mesh=plsc.VectorSubcoreMesh(core_axis_name="c", subcore_axis_name="s"),
        scratch_types=[
            pltpu.VMEM((WIN,), jnp.int32),
            pltpu.VMEM((WIN,), jnp.int32),
            pltpu.VMEM((WIN, H), jnp.float32),
            pltpu.VMEM_SHARED((N, H), jnp.float32),
            pltpu.SemaphoreType.DMA,
        ],
    )
    return f(h, src, dst, zeros).reshape(NC, N, H)


def _dense1_body(sh_ref, h_ref, se_ref,
                 we2_ref, be1_ref, be2_ref,
                 wm1_ref, bm1_ref, wm2_ref, bm2_ref,
                 out_ref, stats_ref, acc_ref):
    i = pl.program_id(0)

    @pl.when(i == 0)
    def _():
        acc_ref[...] = jnp.zeros_like(acc_ref)

    h = h_ref[...]
    sh = sh_ref[0] + sh_ref[1]
    se = se_ref[0, 0] + se_ref[0, 1]
    c0 = (jnp.dot(_leaky(be1_ref[0]), we2_ref[0],
                  preferred_element_type=jnp.float32) + be2_ref[0])
    agg = sh + h + se + c0
    u = jnp.maximum(
        jnp.dot(agg, wm1_ref[0], preferred_element_type=jnp.float32)
        + bm1_ref[0], 0.0)
    h2 = (jnp.dot(u, wm2_ref[0], preferred_element_type=jnp.float32)
          + bm2_ref[0])
    out_ref[...] = h2
    acc_ref[0:1, :] += jnp.sum(h2, axis=0, keepdims=True)
    acc_ref[1:2, :] += jnp.sum(h2 * h2, axis=0, keepdims=True)

    @pl.when(i == _NB - 1)
    def _():
        stats_ref[...] = acc_ref[...]


def _dense2_body(h2_ref, stats_ref, gamma_ref, beta_ref, out_ref, *, last):
    h2 = h2_ref[...]
    mean = stats_ref[0:1, :] * (1.0 / N)
    var = stats_ref[1:2, :] * (1.0 / N) - mean * mean
    out = (h2 - mean) * lax.rsqrt(var + 1e-5) * gamma_ref[0] + beta_ref[0]
    if not last:
        out = jnp.maximum(out, 0.0)
    out_ref[...] = out


def _dense_layer(l, sh, h, se, We2, be1, be2, Wm1, bm1, Wm2, bm2,
                 gamma, beta):
    h2, stats = pl.pallas_call(
        _dense1_body,
        grid=(_NB,),
        in_specs=[
            pl.BlockSpec((NC, _BR, H), lambda i: (0, i, 0)),
            pl.BlockSpec((_BR, H), lambda i: (i, 0)),
            pl.BlockSpec((1, NC, _BR, H), lambda i: (l, 0, i, 0)),
            pl.BlockSpec((1, H, H), lambda i: (l, 0, 0)),
            pl.BlockSpec((1, 1, H), lambda i: (l, 0, 0)),
            pl.BlockSpec((1, 1, H), lambda i: (l, 0, 0)),
            pl.BlockSpec((1, H, 2 * H), lambda i: (l, 0, 0)),
            pl.BlockSpec((1, 1, 2 * H), lambda i: (l, 0, 0)),
            pl.BlockSpec((1, 2 * H, H), lambda i: (l, 0, 0)),
            pl.BlockSpec((1, 1, H), lambda i: (l, 0, 0)),
        ],
        out_specs=[
            pl.BlockSpec((_BR, H), lambda i: (i, 0)),
            pl.BlockSpec((8, H), lambda i: (0, 0)),
        ],
        out_shape=[
            jax.ShapeDtypeStruct((N, H), jnp.float32),
            jax.ShapeDtypeStruct((8, H), jnp.float32),
        ],
        scratch_shapes=[pltpu.VMEM((8, H), jnp.float32)],
    )(sh, h, se, We2, be1, be2, Wm1, bm1, Wm2, bm2)

    body2 = functools.partial(_dense2_body, last=(l == L - 1))
    return pl.pallas_call(
        body2,
        grid=(_NB,),
        in_specs=[
            pl.BlockSpec((_BR, H), lambda i: (i, 0)),
            pl.BlockSpec((8, H), lambda i: (0, 0)),
            pl.BlockSpec((1, 1, H), lambda i: (l, 0, 0)),
            pl.BlockSpec((1, 1, H), lambda i: (l, 0, 0)),
        ],
        out_specs=pl.BlockSpec((_BR, H), lambda i: (i, 0)),
        out_shape=jax.ShapeDtypeStruct((N, H), jnp.float32),
    )(h2, stats, gamma, beta)


def kernel(x, edge_index, edge_attr, W0a, b0a, W0b, b0b, Wm1, bm1, Wm2, bm2,
           We1, be1, We2, be2, gamma, beta):
    src = edge_index[0]
    dst = edge_index[1]

    be1 = be1.reshape(L, 1, H)
    be2 = be2.reshape(L, 1, H)
    bm1 = bm1.reshape(L, 1, 2 * H)
    bm2 = bm2.reshape(L, 1, H)
    gamma = gamma.reshape(L, 1, H)
    beta = beta.reshape(L, 1, H)

    h = _node_mlp(x, W0a, b0a, W0b, b0b)
    e = _edge_mlp(edge_attr, We1, be1, We2, be2)
    se = _sc_edge_agg(e, dst)

    for l in range(L):
        sh = _sc_msg_agg(h, src, dst)
        h = _dense_layer(l, sh, h, se, We2, be1, be2, Wm1, bm1,
                         Wm2, bm2, gamma, beta)
    return h


# speedup vs baseline: 4.7828x; 1.4693x over previous
import functools

import jax
import jax.numpy as jnp
from jax import lax
from jax.experimental import pallas as pl
from jax.experimental.pallas import tpu as pltpu
from jax.experimental.pallas import tpu_sc as plsc

N = 10000
E = 320000
H = 128
L = 3

NC, NS = 2, 16
EPC = E // NC
EPT = E // (NC * NS)
WIN = 40


def _leaky(v):
    return jnp.where(v > 0, v, 0.1 * v)


def _node_mlp_body(x_ref, wa_ref, ba_ref, wb_ref, bb_ref, out_ref):
    t = jnp.dot(x_ref[...], wa_ref[...], preferred_element_type=jnp.float32)
    t = _leaky(t + ba_ref[...])
    out_ref[...] = (
        jnp.dot(t, wb_ref[...], preferred_element_type=jnp.float32) + bb_ref[...]
    )


_NB = 10
_BR = N // _NB


def _node_mlp(x, W0a, b0a, W0b, b0b):
    d_in = x.shape[1]
    return pl.pallas_call(
        _node_mlp_body,
        grid=(_NB,),
        in_specs=[
            pl.BlockSpec((_BR, d_in), lambda i: (i, 0)),
            pl.BlockSpec((d_in, H), lambda i: (0, 0)),
            pl.BlockSpec((1, H), lambda i: (0, 0)),
            pl.BlockSpec((H, H), lambda i: (0, 0)),
            pl.BlockSpec((1, H), lambda i: (0, 0)),
        ],
        out_specs=pl.BlockSpec((_BR, H), lambda i: (i, 0)),
        out_shape=jax.ShapeDtypeStruct((N, H), jnp.float32),
    )(x, W0a, b0a.reshape(1, H), W0b, b0b.reshape(1, H))


_EC = 8000


def _edge_mlp_body(ea_ref, w1_ref, b1_ref, w2_ref, b2_ref, out_ref):
    t = _leaky(
        jnp.dot(ea_ref[...], w1_ref[0], preferred_element_type=jnp.float32)
        + b1_ref[0]
    )
    out_ref[...] = (
        jnp.dot(t, w2_ref[0], preferred_element_type=jnp.float32) + b2_ref[0]
    )


def _edge_mlp(ea, We1, be1, We2, be2):
    d_e = ea.shape[1]
    return pl.pallas_call(
        _edge_mlp_body,
        grid=(L, E // _EC),
        in_specs=[
            pl.BlockSpec((_EC, d_e), lambda l, i: (i, 0)),
            pl.BlockSpec((1, d_e, H), lambda l, i: (l, 0, 0)),
            pl.BlockSpec((1, 1, H), lambda l, i: (l, 0, 0)),
            pl.BlockSpec((1, H, H), lambda l, i: (l, 0, 0)),
            pl.BlockSpec((1, 1, H), lambda l, i: (l, 0, 0)),
        ],
        out_specs=pl.BlockSpec((_EC, H), lambda l, i: (l * (E // _EC) + i, 0)),
        out_shape=jax.ShapeDtypeStruct((L * E, H), jnp.float32),
    )(ea, We1, be1, We2, be2)


ZR = 1000
NZT = N // ZR


KG = 5
GWIN = KG * WIN
NG = EPT // GWIN


def _sc_edge_agg_body(e_hbm, dst_hbm, zeros_hbm,
                      se_hbm,
                      didx0, didx1, didx2, didx3, didx4,
                      rows, acc_sh, isem, ssem):
    c = lax.axis_index("c")
    s = lax.axis_index("s")
    base = c * EPC + s * EPT
    r0 = s * ZR
    didx = (didx0, didx1, didx2, didx3, didx4)

    for l in range(L):
        @pl.when(s < NZT)
        def _():
            pltpu.sync_copy(zeros_hbm.at[pl.ds(r0, ZR)],
                            acc_sh.at[pl.ds(r0, ZR)])

        plsc.subcore_barrier()

        def grp_body(g, carry):
            goff = base + g * GWIN
            cps = [pltpu.async_copy(dst_hbm.at[pl.ds(goff + k * WIN, WIN)],
                                    didx[k], isem) for k in range(KG)]
            pltpu.sync_copy(e_hbm.at[pl.ds(l * E + goff, GWIN)], rows)
            for cp in cps:
                cp.wait()
            sps = [pltpu.async_copy(rows.at[pl.ds(k * WIN, WIN)],
                                    acc_sh.at[didx[k]], ssem, add=True)
                   for k in range(KG)]
            for sp in sps:
                sp.wait()
            return carry

        lax.fori_loop(0, NG, grp_body, 0)
        plsc.subcore_barrier()

        @pl.when(s < NZT)
        def _():
            pltpu.sync_copy(acc_sh.at[pl.ds(r0, ZR)],
                            se_hbm.at[pl.ds((l * NC + c) * N + r0, ZR)])

        plsc.subcore_barrier()


def _sc_edge_agg(e, dst):
    zeros = jnp.zeros((N, H), jnp.float32)
    f = pl.kernel(
        _sc_edge_agg_body,
        out_type=jax.ShapeDtypeStruct((L * NC * N, H), jnp.float32),
        mesh=plsc.VectorSubcoreMesh(core_axis_name="c", subcore_axis_name="s"),
        scratch_types=(
            [pltpu.VMEM((WIN,), jnp.int32) for _ in range(5)]
            + [
                pltpu.VMEM((GWIN, H), jnp.float32),
                pltpu.VMEM_SHARED((N, H), jnp.float32),
                pltpu.SemaphoreType.DMA,
                pltpu.SemaphoreType.DMA,
            ]
        ),
    )
    return f(e, dst, zeros).reshape(L, NC, N, H)


def _sc_msg_agg_body(h_hbm, src_hbm, dst_hbm, zeros_hbm,
                     sh_hbm,
                     sidx0, sidx1, sidx2, sidx3, sidx4,
                     didx0, didx1, didx2, didx3, didx4,
                     rows, acc_sh, isem, gsem, ssem):
    c = lax.axis_index("c")
    s = lax.axis_index("s")
    base = c * EPC + s * EPT
    r0 = s * ZR
    sidx = (sidx0, sidx1, sidx2, sidx3, sidx4)
    didx = (didx0, didx1, didx2, didx3, didx4)

    @pl.when(s < NZT)
    def _():
        pltpu.sync_copy(zeros_hbm.at[pl.ds(r0, ZR)],
                        acc_sh.at[pl.ds(r0, ZR)])

    plsc.subcore_barrier()

    def grp_body(g, carry):
        goff = base + g * GWIN
        cps = [pltpu.async_copy(src_hbm.at[pl.ds(goff + k * WIN, WIN)],
                                sidx[k], isem) for k in range(KG)]
        cps += [pltpu.async_copy(dst_hbm.at[pl.ds(goff + k * WIN, WIN)],
                                 didx[k], isem) for k in range(KG)]
        for cp in cps:
            cp.wait()
        gps = [pltpu.async_copy(h_hbm.at[sidx[k]],
                                rows.at[pl.ds(k * WIN, WIN)], gsem)
               for k in range(KG)]
        for gp in gps:
            gp.wait()
        sps = [pltpu.async_copy(rows.at[pl.ds(k * WIN, WIN)],
                                acc_sh.at[didx[k]], ssem, add=True)
               for k in range(KG)]
        for sp in sps:
            sp.wait()
        return carry

    lax.fori_loop(0, NG, grp_body, 0)
    plsc.subcore_barrier()

    @pl.when(s < NZT)
    def _():
        pltpu.sync_copy(acc_sh.at[pl.ds(r0, ZR)],
                        sh_hbm.at[pl.ds(c * N + r0, ZR)])


def _sc_msg_agg(h, src, dst):
    zeros = jnp.zeros((N, H), jnp.float32)
    f = pl.kernel(
        _sc_msg_agg_body,
        out_type=jax.ShapeDtypeStruct((NC * N, H), jnp.float32),
        mesh=plsc.VectorSubcoreMesh(core_axis_name="c", subcore_axis_name="s"),
        scratch_types=(
            [pltpu.VMEM((WIN,), jnp.int32) for _ in range(10)]
            + [
                pltpu.VMEM((GWIN, H), jnp.float32),
                pltpu.VMEM_SHARED((N, H), jnp.float32),
                pltpu.SemaphoreType.DMA,
                pltpu.SemaphoreType.DMA,
                pltpu.SemaphoreType.DMA,
            ]
        ),
    )
    return f(h, src, dst, zeros).reshape(NC, N, H)


def _dense1_body(sh_ref, h_ref, se_ref,
                 we2_ref, be1_ref, be2_ref,
                 wm1_ref, bm1_ref, wm2_ref, bm2_ref,
                 out_ref, stats_ref, acc_ref):
    i = pl.program_id(0)

    @pl.when(i == 0)
    def _():
        acc_ref[...] = jnp.zeros_like(acc_ref)

    h = h_ref[...]
    sh = sh_ref[0] + sh_ref[1]
    se = se_ref[0, 0] + se_ref[0, 1]
    c0 = (jnp.dot(_leaky(be1_ref[0]), we2_ref[0],
                  preferred_element_type=jnp.float32) + be2_ref[0])
    agg = sh + h + se + c0
    u = jnp.maximum(
        jnp.dot(agg, wm1_ref[0], preferred_element_type=jnp.float32)
        + bm1_ref[0], 0.0)
    h2 = (jnp.dot(u, wm2_ref[0], preferred_element_type=jnp.float32)
          + bm2_ref[0])
    out_ref[...] = h2
    acc_ref[0:1, :] += jnp.sum(h2, axis=0, keepdims=True)
    acc_ref[1:2, :] += jnp.sum(h2 * h2, axis=0, keepdims=True)

    @pl.when(i == _NB - 1)
    def _():
        stats_ref[...] = acc_ref[...]


def _dense2_body(h2_ref, stats_ref, gamma_ref, beta_ref, out_ref, *, last):
    h2 = h2_ref[...]
    mean = stats_ref[0:1, :] * (1.0 / N)
    var = stats_ref[1:2, :] * (1.0 / N) - mean * mean
    out = (h2 - mean) * lax.rsqrt(var + 1e-5) * gamma_ref[0] + beta_ref[0]
    if not last:
        out = jnp.maximum(out, 0.0)
    out_ref[...] = out


def _dense_layer(l, sh, h, se, We2, be1, be2, Wm1, bm1, Wm2, bm2,
                 gamma, beta):
    h2, stats = pl.pallas_call(
        _dense1_body,
        grid=(_NB,),
        in_specs=[
            pl.BlockSpec((NC, _BR, H), lambda i: (0, i, 0)),
            pl.BlockSpec((_BR, H), lambda i: (i, 0)),
            pl.BlockSpec((1, NC, _BR, H), lambda i: (l, 0, i, 0)),
            pl.BlockSpec((1, H, H), lambda i: (l, 0, 0)),
            pl.BlockSpec((1, 1, H), lambda i: (l, 0, 0)),
            pl.BlockSpec((1, 1, H), lambda i: (l, 0, 0)),
            pl.BlockSpec((1, H, 2 * H), lambda i: (l, 0, 0)),
            pl.BlockSpec((1, 1, 2 * H), lambda i: (l, 0, 0)),
            pl.BlockSpec((1, 2 * H, H), lambda i: (l, 0, 0)),
            pl.BlockSpec((1, 1, H), lambda i: (l, 0, 0)),
        ],
        out_specs=[
            pl.BlockSpec((_BR, H), lambda i: (i, 0)),
            pl.BlockSpec((8, H), lambda i: (0, 0)),
        ],
        out_shape=[
            jax.ShapeDtypeStruct((N, H), jnp.float32),
            jax.ShapeDtypeStruct((8, H), jnp.float32),
        ],
        scratch_shapes=[pltpu.VMEM((8, H), jnp.float32)],
    )(sh, h, se, We2, be1, be2, Wm1, bm1, Wm2, bm2)

    body2 = functools.partial(_dense2_body, last=(l == L - 1))
    return pl.pallas_call(
        body2,
        grid=(_NB,),
        in_specs=[
            pl.BlockSpec((_BR, H), lambda i: (i, 0)),
            pl.BlockSpec((8, H), lambda i: (0, 0)),
            pl.BlockSpec((1, 1, H), lambda i: (l, 0, 0)),
            pl.BlockSpec((1, 1, H), lambda i: (l, 0, 0)),
        ],
        out_specs=pl.BlockSpec((_BR, H), lambda i: (i, 0)),
        out_shape=jax.ShapeDtypeStruct((N, H), jnp.float32),
    )(h2, stats, gamma, beta)


def kernel(x, edge_index, edge_attr, W0a, b0a, W0b, b0b, Wm1, bm1, Wm2, bm2,
           We1, be1, We2, be2, gamma, beta):
    src = edge_index[0]
    dst = edge_index[1]

    be1 = be1.reshape(L, 1, H)
    be2 = be2.reshape(L, 1, H)
    bm1 = bm1.reshape(L, 1, 2 * H)
    bm2 = bm2.reshape(L, 1, H)
    gamma = gamma.reshape(L, 1, H)
    beta = beta.reshape(L, 1, H)

    h = _node_mlp(x, W0a, b0a, W0b, b0b)
    e = _edge_mlp(edge_attr, We1, be1, We2, be2)
    se = _sc_edge_agg(e, dst)

    for l in range(L):
        sh = _sc_msg_agg(h, src, dst)
        h = _dense_layer(l, sh, h, se, We2, be1, be2, Wm1, bm1,
                         Wm2, bm2, gamma, beta)
    return h
